# Initial kernel scaffold; baseline (speedup 1.0000x reference)
import jax
import jax.numpy as jnp
from jax.experimental import pallas as pl


def kernel(x, labels, feats):
    raise NotImplementedError("write your pallas kernel here")


# speedup vs baseline: 1.0592x; 1.0592x over previous
import functools

import jax
import jax.numpy as jnp
from jax.experimental import pallas as pl
from jax.experimental.pallas import tpu as pltpu

_NUM_PIDS = 100000
_BATCH = 1024
_FEAT = 128
_BLOCK_C = 2000
_GRID = _NUM_PIDS // _BLOCK_C


def _fused_kernel(x_ref, labels_ref, feats_ref, out_ref, min_acc, own_acc,
                  *, block_c, grid):
    step = pl.program_id(0)
    x = x_ref[...]
    f = feats_ref[...]

    xb = x.astype(jnp.bfloat16)
    fb = f.astype(jnp.bfloat16)
    prod = jax.lax.dot_general(
        xb, fb, (((1,), (1,)), ((), ())),
        preferred_element_type=jnp.float32)
    ones_row = jnp.ones((1, f.shape[1]), jnp.float32)
    yy = jax.lax.dot_general(
        ones_row, f * f, (((1,), (1,)), ((), ())),
        preferred_element_type=jnp.float32,
        precision=jax.lax.Precision.HIGHEST)
    s = yy - 2.0 * prod

    labels = labels_ref[...]
    col = jax.lax.broadcasted_iota(jnp.int32, s.shape, 1)
    own_mask = (labels - step * block_c) == col

    bmin = jnp.min(jnp.where(own_mask, jnp.inf, s), axis=1, keepdims=True)
    bown = jnp.sum(jnp.where(own_mask, s, 0.0), axis=1, keepdims=True)

    @pl.when(step == 0)
    def _init():
        min_acc[...] = bmin
        own_acc[...] = bown

    @pl.when(step > 0)
    def _update():
        min_acc[...] = jnp.minimum(min_acc[...], bmin)
        own_acc[...] = own_acc[...] + bown

    @pl.when(step == grid - 1)
    def _finish():
        xx = jnp.sum(x * x, axis=1, keepdims=True)
        d_an = jnp.sqrt(jnp.clip(xx + min_acc[...], 1e-12, None))
        d_ap = jnp.sqrt(jnp.clip(xx + own_acc[...], 1e-12, None))
        loss = jnp.mean(jnp.logaddexp(0.0, d_ap - d_an))
        out_ref[...] = loss[None, None]


def kernel(x, labels, feats):
    labels2d = labels.reshape(_BATCH, 1).astype(jnp.int32)
    out = pl.pallas_call(
        functools.partial(_fused_kernel, block_c=_BLOCK_C, grid=_GRID),
        grid=(_GRID,),
        in_specs=[
            pl.BlockSpec((_BATCH, _FEAT), lambda i: (0, 0)),
            pl.BlockSpec((_BATCH, 1), lambda i: (0, 0)),
            pl.BlockSpec((_BLOCK_C, _FEAT), lambda i: (i, 0)),
        ],
        out_specs=pl.BlockSpec((1, 1), lambda i: (0, 0)),
        out_shape=jax.ShapeDtypeStruct((1, 1), jnp.float32),
        scratch_shapes=[
            pltpu.VMEM((_BATCH, 1), jnp.float32),
            pltpu.VMEM((_BATCH, 1), jnp.float32),
        ],
        compiler_params=pltpu.CompilerParams(
            dimension_semantics=("arbitrary",),
        ),
    )(x, labels2d, feats)
    return out[0, 0]


# speedup vs baseline: 1.7835x; 1.6837x over previous
import functools

import jax
import jax.numpy as jnp
from jax.experimental import pallas as pl
from jax.experimental.pallas import tpu as pltpu

_NUM_PIDS = 100000
_BATCH = 1024
_FEAT = 128
_BLOCK_C = 2000
_GRID = _NUM_PIDS // _BLOCK_C


def _fused_kernel(x_ref, labels_ref, feats_ref, out_ref, min_acc, own_acc,
                  *, block_c, grid):
    step = pl.program_id(0)
    x = x_ref[...]
    labels = labels_ref[...]
    f = feats_ref[...]

    fb = f.astype(jnp.bfloat16)
    yy = jnp.sum(f * f, axis=1, keepdims=True)
    yhi = yy.astype(jnp.bfloat16)
    ylo = (yy - yhi.astype(jnp.float32)).astype(jnp.bfloat16)
    f_aug = jnp.concatenate([fb, yhi, ylo], axis=1)
    xm2 = (-2.0 * x).astype(jnp.bfloat16)
    ones2 = jnp.ones((x.shape[0], 2), jnp.bfloat16)
    x_aug = jnp.concatenate([xm2, ones2], axis=1)
    s = jax.lax.dot_general(
        x_aug, f_aug, (((1,), (1,)), ((), ())),
        preferred_element_type=jnp.float32)

    col = jax.lax.broadcasted_iota(jnp.int32, s.shape, 1)
    rel = labels - step * block_c
    bmin = jnp.min(jnp.where(rel == col, jnp.inf, s), axis=1, keepdims=True)
    bown = jnp.min(jnp.where(rel != col, jnp.inf, s), axis=1, keepdims=True)

    @pl.when(step == 0)
    def _init():
        min_acc[...] = bmin
        own_acc[...] = bown

    @pl.when(step > 0)
    def _update():
        min_acc[...] = jnp.minimum(min_acc[...], bmin)
        own_acc[...] = jnp.minimum(own_acc[...], bown)

    @pl.when(step == grid - 1)
    def _finish():
        xx = jnp.sum(x * x, axis=1, keepdims=True)
        d_an = jnp.sqrt(jnp.clip(xx + min_acc[...], 1e-12, None))
        d_ap = jnp.sqrt(jnp.clip(xx + own_acc[...], 1e-12, None))
        loss = jnp.mean(jnp.logaddexp(0.0, d_ap - d_an))
        out_ref[...] = loss[None, None]


def kernel(x, labels, feats):
    labels2d = labels.reshape(_BATCH, 1).astype(jnp.int32)
    out = pl.pallas_call(
        functools.partial(_fused_kernel, block_c=_BLOCK_C, grid=_GRID),
        grid=(_GRID,),
        in_specs=[
            pl.BlockSpec((_BATCH, _FEAT), lambda i: (0, 0)),
            pl.BlockSpec((_BATCH, 1), lambda i: (0, 0)),
            pl.BlockSpec((_BLOCK_C, _FEAT), lambda i: (i, 0)),
        ],
        out_specs=pl.BlockSpec((1, 1), lambda i: (0, 0)),
        out_shape=jax.ShapeDtypeStruct((1, 1), jnp.float32),
        scratch_shapes=[
            pltpu.VMEM((_BATCH, 1), jnp.float32),
            pltpu.VMEM((_BATCH, 1), jnp.float32),
        ],
        compiler_params=pltpu.CompilerParams(
            dimension_semantics=("arbitrary",),
        ),
    )(x, labels2d, feats)
    return out[0, 0]
